# Initial kernel scaffold; baseline (speedup 1.0000x reference)
import jax
import jax.numpy as jnp
from jax.experimental import pallas as pl


def kernel(features, W_user, W_ad, W_loc, W_cat, W_sq, W_sp, W_title, W_params):
    raise NotImplementedError("write your pallas kernel here")


# speedup vs baseline: 4.9858x; 4.9858x over previous
import functools

import jax
import jax.numpy as jnp
from jax import lax
from jax.experimental import pallas as pl
from jax.experimental.pallas import tpu as pltpu
from jax.experimental.pallas import tpu_sc as plsc

B = 16384
D = 16
NSLOT = 28
NC, NS, L = 2, 16, 16
NW = NC * NS
ROWS_PER_W = B // NW
CHUNK = 64
NCHUNK = ROWS_PER_W // CHUNK

V = 10000
_SINGLE = (
    [(0, c, c * V) for c in range(8)]
    + [(1, fc, j * V) for j, fc in enumerate((8, 9, 10, 23, 37, 38, 39))]
    + [(2, 11 + j, j * V) for j in range(3)]
    + [(3, 14 + j, j * V) for j in range(3)]
    + [(4, 17, 0)]
    + [(3, 24 + j, j * V) for j in range(3)]
)
_SINGLE_SLOT = list(range(22)) + [23, 24, 25]
_SUMMED = (
    (5, (18, 19, 20, 21, 22), 22),
    (6, (27, 28, 29, 30, 31), 26),
    (7, (32, 33, 34, 35, 36), 27),
)
_GATHERS = list(_SINGLE)
for t, cols, _slot in _SUMMED:
    _GATHERS.extend((t, c, 0) for c in cols)
assert len(_GATHERS) == 40

_mesh = plsc.VectorSubcoreMesh(core_axis_name="c", subcore_axis_name="s")


@functools.partial(
    pl.kernel,
    out_type=jax.ShapeDtypeStruct((B * NSLOT, D), jnp.float32),
    mesh=_mesh,
    scratch_types=[
        pltpu.VMEM((2, 40, CHUNK), jnp.int32),
        pltpu.VMEM((2, 40, CHUNK), jnp.int32),
        pltpu.VMEM((2, NSLOT, CHUNK), jnp.int32),
        pltpu.VMEM((2 * 40 * CHUNK, D), jnp.float32),
        pltpu.VMEM((2 * 3 * CHUNK, D), jnp.float32),
        pltpu.SemaphoreType.DMA,
        pltpu.SemaphoreType.DMA,
    ],
    compiler_params=pltpu.CompilerParams(use_tc_tiling_on_sc=False),
)
def _iicn_sc(featT, W_user, W_ad, W_loc, W_cat, W_sq, W_sp, W_title,
             W_params, out, feat_v, idx_v, widx_v, rows_v, sums_v,
             sem_g, sem_s):
    tables = (W_user, W_ad, W_loc, W_cat, W_sq, W_sp, W_title, W_params)
    wid = lax.axis_index("s") * NC + lax.axis_index("c")
    base0 = wid * ROWS_PER_W
    lanes = lax.iota(jnp.int32, L)
    lanes28 = lanes * NSLOT

    def load_and_index(buf, ci):
        base = base0 + ci * CHUNK
        pltpu.sync_copy(featT.at[:, pl.ds(base, CHUNK)], feat_v.at[buf])

        @pl.loop(0, CHUNK, step=L)
        def _ib(q0):
            for g, (t, col, off) in enumerate(_GATHERS):
                idx_v.at[buf].at[g][pl.ds(q0, L)] = (
                    feat_v.at[buf].at[col][pl.ds(q0, L)] + off)
            b28 = (base + q0) * NSLOT + lanes28
            for s in range(NSLOT):
                widx_v.at[buf].at[s][pl.ds(q0, L)] = b28 + s

    def gather_copies(buf):
        cps = []
        for g, (t, col, off) in enumerate(_GATHERS):
            dst = rows_v.at[pl.ds((buf * 40 + g) * CHUNK, CHUNK)]
            cps.append(pltpu.make_async_copy(
                tables[t].at[idx_v.at[buf].at[g]], dst, sem_g))
        return cps

    def scatter_copies(buf):
        cps = []
        for s in range(NSLOT):
            if s == 22:
                src = sums_v.at[pl.ds((buf * 3 + 0) * CHUNK, CHUNK)]
            elif s == 26:
                src = sums_v.at[pl.ds((buf * 3 + 1) * CHUNK, CHUNK)]
            elif s == 27:
                src = sums_v.at[pl.ds((buf * 3 + 2) * CHUNK, CHUNK)]
            else:
                g = _SINGLE_SLOT.index(s)
                src = rows_v.at[pl.ds((buf * 40 + g) * CHUNK, CHUNK)]
            cps.append(pltpu.make_async_copy(
                src, out.at[widx_v.at[buf].at[s]], sem_s))
        return cps

    def fire(cps):
        for cp in cps:
            cp.start()

    def drain(cps):
        for cp in cps:
            cp.wait()

    def sums(buf):
        for grp in range(3):
            first = (buf * 40 + 25 + grp * 5) * CHUNK

            @pl.loop(0, CHUNK)
            def _sum(b, _first=first, _grp=grp, _buf=buf):
                acc = rows_v[_first + b, :]
                for j in range(1, 5):
                    acc = acc + rows_v[_first + j * CHUNK + b, :]
                sums_v.at[(_buf * 3 + _grp) * CHUNK + b][:] = acc

    load_and_index(0, 0)
    fire(gather_copies(0))
    drain(gather_copies(0))
    load_and_index(1, 1)
    fire(gather_copies(1))
    sums(0)
    fire(scatter_copies(0))

    @pl.loop(0, (NCHUNK - 2) // 2)
    def _pipe(k):
        for phase in range(2):
            ci = 2 * k + 1 + phase
            buf = 1 - phase
            drain(gather_copies(buf))
            drain(scatter_copies(buf ^ 1))
            load_and_index(buf ^ 1, ci + 1)
            fire(gather_copies(buf ^ 1))
            sums(buf)
            fire(scatter_copies(buf))

    drain(gather_copies(1))
    drain(scatter_copies(0))
    sums(1)
    fire(scatter_copies(1))
    drain(scatter_copies(1))


def kernel(features, W_user, W_ad, W_loc, W_cat, W_sq, W_sp, W_title,
           W_params):
    out = _iicn_sc(
        features.T,
        W_user[:, :V, :].reshape(8 * V, D),
        W_ad[:, :V, :].reshape(8 * V, D),
        W_loc.reshape(3 * V, D),
        W_cat.reshape(3 * V, D),
        W_sq[:V], W_sp[:V], W_title[:V], W_params[:V],
    )
    return out.reshape(B, NSLOT * D)
